# baseline (speedup 1.0000x reference)
import functools

import jax
import jax.numpy as jnp
from jax import lax
from jax.experimental import pallas as pl
from jax.experimental.pallas import tpu as pltpu
from jax.experimental.pallas import tpu_sc as plsc

NUM_CLASSES = 1000000
LAT_DIM = 32
BATCH = 16384

_INFO = plsc.get_sparse_core_info()
_NC, _NS, _L = _INFO.num_cores, _INFO.num_subcores, _INFO.num_lanes
_NW = _NC * _NS
_BPW = BATCH // _NW
_CHUNK = 128
_NCHUNK = _BPW // _CHUNK


def _body(idx_hbm, tab_hbm, mu_hbm, sig_hbm, idx_v, dst_v, sem):
    wid = lax.axis_index("s") * _NC + lax.axis_index("c")
    base = wid * _BPW

    pltpu.sync_copy(idx_hbm.at[pl.ds(base, _BPW)], idx_v)

    copies = [
        pltpu.async_copy(
            tab_hbm.at[idx_v.at[pl.ds(j * _CHUNK, _CHUNK)]],
            dst_v.at[pl.ds(j * _CHUNK, _CHUNK)], sem)
        for j in range(_NCHUNK)
    ]
    for c in copies:
        c.wait()

    def exp_rows(i, _):
        r0 = i * 2
        for r in range(2):
            dst_v[r0 + r, LAT_DIM:LAT_DIM + _L] = jnp.exp(
                dst_v[r0 + r, LAT_DIM:LAT_DIM + _L])
            dst_v[r0 + r, LAT_DIM + _L:2 * LAT_DIM] = jnp.exp(
                dst_v[r0 + r, LAT_DIM + _L:2 * LAT_DIM])
        return _

    lax.fori_loop(0, _BPW // 2, exp_rows, None)

    pltpu.sync_copy(dst_v.at[:, 0:LAT_DIM], mu_hbm.at[pl.ds(base, _BPW)])
    pltpu.sync_copy(dst_v.at[:, LAT_DIM:2 * LAT_DIM],
                    sig_hbm.at[pl.ds(base, _BPW)])


@jax.jit
def _prior_sc(indices, table):
    f32 = jnp.float32
    run = functools.partial(
        pl.kernel,
        out_type=(jax.ShapeDtypeStruct((BATCH, LAT_DIM), f32),
                  jax.ShapeDtypeStruct((BATCH, LAT_DIM), f32)),
        mesh=plsc.VectorSubcoreMesh(core_axis_name="c", subcore_axis_name="s"),
        compiler_params=pltpu.CompilerParams(use_tc_tiling_on_sc=False),
        scratch_types=[
            pltpu.VMEM((_BPW,), jnp.int32),
            pltpu.VMEM((_BPW, 2 * LAT_DIM), f32),
            pltpu.SemaphoreType.DMA,
        ],
    )(_body)
    return run(indices, table)


def kernel(indices, table):
    mu, sigma = _prior_sc(indices.astype(jnp.int32), table)
    return (mu, sigma)


# speedup vs baseline: 1.5482x; 1.5482x over previous
import functools

import jax
import jax.numpy as jnp
from jax import lax
from jax.experimental import pallas as pl
from jax.experimental.pallas import tpu as pltpu
from jax.experimental.pallas import tpu_sc as plsc

NUM_CLASSES = 1000000
LAT_DIM = 32
BATCH = 16384

_INFO = plsc.get_sparse_core_info()
_NC, _NS, _L = _INFO.num_cores, _INFO.num_subcores, _INFO.num_lanes
_NW = _NC * _NS
_BPW = BATCH // _NW
_CHUNK = 64
_NCHUNK = _BPW // _CHUNK
_GRP = 8


def _body(idx_hbm, tab_hbm, mut_hbm, sigt_hbm,
          idx_v, rows_v, mut_v, sigt_v, sem):
    wid = lax.axis_index("s") * _NC + lax.axis_index("c")
    base = wid * _BPW

    pltpu.sync_copy(idx_hbm.at[pl.ds(base, _BPW)], idx_v)

    lanes = lax.iota(jnp.int32, _L)

    def do_chunk(j, _):
        def fire(i16, _):
            v = idx_v[pl.ds(j * _CHUNK + i16 * _L, _L)]
            for s in range(_L):
                r8 = pl.multiple_of((v[s] >> 3) * _GRP, _GRP)
                pltpu.async_copy(
                    tab_hbm.at[pl.ds(r8, _GRP), :],
                    rows_v.at[pl.ds((i16 * _L + s) * _GRP, _GRP), :], sem)
            return _

        lax.fori_loop(0, _CHUNK // _L, fire, None)

        def drain(i, _):
            pltpu.make_async_copy(
                tab_hbm.at[pl.ds(0, _GRP), :],
                rows_v.at[pl.ds(0, _GRP), :], sem).wait()
            return _

        lax.fori_loop(0, _CHUNK, drain, None)

        def select(g, _):
            o = j * _CHUNK + g * _L
            loc = idx_v[pl.ds(o, _L)] & 7
            row16 = (g * _L + lanes) * _GRP + loc
            for f in range(LAT_DIM):
                mut_v[f, pl.ds(o, _L)] = plsc.load_gather(
                    rows_v, [row16, jnp.full((_L,), f, jnp.int32)])
            for f in range(LAT_DIM):
                sigt_v[f, pl.ds(o, _L)] = jnp.exp(plsc.load_gather(
                    rows_v, [row16, jnp.full((_L,), LAT_DIM + f, jnp.int32)]))
            return _

        lax.fori_loop(0, _CHUNK // _L, select, None)
        return _

    lax.fori_loop(0, _NCHUNK, do_chunk, None)

    pltpu.sync_copy(mut_v, mut_hbm.at[:, pl.ds(base, _BPW)])
    pltpu.sync_copy(sigt_v, sigt_hbm.at[:, pl.ds(base, _BPW)])


@jax.jit
def _prior_sc(indices, table):
    f32 = jnp.float32
    run = functools.partial(
        pl.kernel,
        out_type=(jax.ShapeDtypeStruct((LAT_DIM, BATCH), f32),
                  jax.ShapeDtypeStruct((LAT_DIM, BATCH), f32)),
        mesh=plsc.VectorSubcoreMesh(core_axis_name="c", subcore_axis_name="s"),
        compiler_params=pltpu.CompilerParams(needs_layout_passes=False),
        scratch_types=[
            pltpu.VMEM((_BPW,), jnp.int32),
            pltpu.VMEM((_CHUNK * _GRP, 64), f32),
            pltpu.VMEM((LAT_DIM, _BPW), f32),
            pltpu.VMEM((LAT_DIM, _BPW), f32),
            pltpu.SemaphoreType.DMA,
        ],
    )(_body)
    return run(indices, table)


def kernel(indices, table):
    mu_t, sigma_t = _prior_sc(indices.astype(jnp.int32), table)
    return (mu_t.T, sigma_t.T)
